# baseline (device time: 890393 ns/iter reference)
import jax
import jax.numpy as jnp
from jax import lax
from jax.experimental import pallas as pl
from jax.experimental.pallas import tpu as pltpu

N_DEV = 32
M = 4096
N = 2048
CH = M // N_DEV
H = N // 2
SLOTS = 4
STEPS = N_DEV - 1


def kernel(x, w_mat):
    partial = jnp.dot(x, w_mat, preferred_element_type=jnp.float32)

    def body(
        p_ref,
        out_ref,
        comm_r,
        comm_l,
        stage_r,
        stage_l,
        local_sems,
        send_sems_r,
        recv_sems_r,
        credit_r,
        send_sems_l,
        recv_sems_l,
        credit_l,
        ag_send_r,
        ag_recv_r,
        ag_credit_r,
        ag_send_l,
        ag_recv_l,
        ag_credit_l,
    ):
        my = lax.axis_index("i")
        left = lax.rem(my - 1 + N_DEV, N_DEV)
        right = lax.rem(my + 1, N_DEV)

        barrier_sem = pltpu.get_barrier_semaphore()
        for nbr in (left, right):
            pl.semaphore_signal(
                barrier_sem,
                inc=1,
                device_id=(nbr,),
                device_id_type=pl.DeviceIdType.MESH,
            )
        pl.semaphore_wait(barrier_sem, 2)

        def rhalf(ref, c):
            return ref.at[pl.ds(c * CH, CH), pl.ds(0, H)]

        def lhalf(ref, c):
            return ref.at[pl.ds(c * CH, CH), pl.ds(H, H)]

        def remote(src, dst, ssem, rsem, dev):
            return pltpu.make_async_remote_copy(
                src_ref=src,
                dst_ref=dst,
                send_sem=ssem,
                recv_sem=rsem,
                device_id=(dev,),
                device_id_type=pl.DeviceIdType.MESH,
            )

        def credit_to(sems, slot, dev):
            pl.semaphore_signal(
                sems.at[slot],
                inc=1,
                device_id=(dev,),
                device_id_type=pl.DeviceIdType.MESH,
            )

        seed = pltpu.make_async_copy(
            p_ref.at[pl.ds(my * CH, CH), :],
            out_ref.at[pl.ds(my * CH, CH), :],
            local_sems.at[0],
        )
        seed.start()
        seed.wait()

        def stage_issue(s):
            rc_r = lax.rem(my - s - 1 + 2 * N_DEV, N_DEV)
            rc_l = lax.rem(my + s + 1, N_DEV)
            sl = s % 2
            cr = pltpu.make_async_copy(
                rhalf(p_ref, rc_r), stage_r.at[sl], local_sems.at[sl]
            )
            cl = pltpu.make_async_copy(
                lhalf(p_ref, rc_l), stage_l.at[sl], local_sems.at[2 + sl]
            )
            cr.start()
            cl.start()
            return cr, cl

        stages = {0: stage_issue(0)}

        sends_r, sends_l = [], []
        for s in range(STEPS):
            slot = s % SLOTS
            sc_r = lax.rem(my - s + N_DEV, N_DEV)
            rc_r = lax.rem(my - s - 1 + 2 * N_DEV, N_DEV)
            sc_l = lax.rem(my + s, N_DEV)
            rc_l = lax.rem(my + s + 1, N_DEV)
            if s >= SLOTS:
                pl.semaphore_wait(credit_r.at[slot], 1)
                pl.semaphore_wait(credit_l.at[slot], 1)
                sends_r[s - SLOTS].wait_send()
                sends_l[s - SLOTS].wait_send()
            rdma_r = remote(
                rhalf(out_ref, sc_r),
                comm_r.at[slot],
                send_sems_r.at[slot],
                recv_sems_r.at[slot],
                right,
            )
            rdma_l = remote(
                lhalf(out_ref, sc_l),
                comm_l.at[slot],
                send_sems_l.at[slot],
                recv_sems_l.at[slot],
                left,
            )
            rdma_r.start()
            rdma_l.start()
            sends_r.append(rdma_r)
            sends_l.append(rdma_l)
            if s + 1 < STEPS:
                stages[s + 1] = stage_issue(s + 1)
            st_r, st_l = stages.pop(s)
            rdma_r.wait_recv()
            st_r.wait()
            acc = stage_r[s % 2] + comm_r[slot]
            if s == STEPS - 1:
                acc = acc * jax.nn.sigmoid(acc)
            out_ref[pl.ds(rc_r * CH, CH), pl.ds(0, H)] = acc
            credit_to(credit_r, slot, left)
            rdma_l.wait_recv()
            st_l.wait()
            acc = stage_l[s % 2] + comm_l[slot]
            if s == STEPS - 1:
                acc = acc * jax.nn.sigmoid(acc)
            out_ref[pl.ds(rc_l * CH, CH), pl.ds(H, H)] = acc
            credit_to(credit_l, slot, right)
        for j in range(SLOTS):
            pl.semaphore_wait(credit_r.at[j], 1)
            pl.semaphore_wait(credit_l.at[j], 1)
        for d in sends_r[-SLOTS:] + sends_l[-SLOTS:]:
            d.wait_send()

        sends_r, sends_l = [], []
        for s in range(STEPS):
            slot = s % SLOTS
            sc_r = lax.rem(my + 1 - s + 2 * N_DEV, N_DEV)
            sc_l = lax.rem(my - 1 + s + N_DEV, N_DEV)
            if s >= SLOTS:
                pl.semaphore_wait(ag_credit_r.at[slot], 1)
                pl.semaphore_wait(ag_credit_l.at[slot], 1)
                sends_r[s - SLOTS].wait_send()
                sends_l[s - SLOTS].wait_send()
            rdma_r = remote(
                rhalf(out_ref, sc_r),
                rhalf(out_ref, sc_r),
                ag_send_r.at[slot],
                ag_recv_r.at[slot],
                right,
            )
            rdma_l = remote(
                lhalf(out_ref, sc_l),
                lhalf(out_ref, sc_l),
                ag_send_l.at[slot],
                ag_recv_l.at[slot],
                left,
            )
            rdma_r.start()
            rdma_l.start()
            sends_r.append(rdma_r)
            sends_l.append(rdma_l)
            rdma_r.wait_recv()
            credit_to(ag_credit_r, slot, left)
            rdma_l.wait_recv()
            credit_to(ag_credit_l, slot, right)
        for j in range(SLOTS):
            pl.semaphore_wait(ag_credit_r.at[j], 1)
            pl.semaphore_wait(ag_credit_l.at[j], 1)
        for d in sends_r[-SLOTS:] + sends_l[-SLOTS:]:
            d.wait_send()

    return pl.pallas_call(
        body,
        out_shape=jax.ShapeDtypeStruct((M, N), jnp.float32),
        in_specs=[pl.BlockSpec(memory_space=pltpu.MemorySpace.HBM)],
        out_specs=pl.BlockSpec(memory_space=pltpu.VMEM),
        scratch_shapes=[
            pltpu.VMEM((SLOTS, CH, H), jnp.float32),
            pltpu.VMEM((SLOTS, CH, H), jnp.float32),
            pltpu.VMEM((2, CH, H), jnp.float32),
            pltpu.VMEM((2, CH, H), jnp.float32),
            pltpu.SemaphoreType.DMA((4,)),
            pltpu.SemaphoreType.DMA((SLOTS,)),
            pltpu.SemaphoreType.DMA((SLOTS,)),
            pltpu.SemaphoreType.REGULAR((SLOTS,)),
            pltpu.SemaphoreType.DMA((SLOTS,)),
            pltpu.SemaphoreType.DMA((SLOTS,)),
            pltpu.SemaphoreType.REGULAR((SLOTS,)),
            pltpu.SemaphoreType.DMA((SLOTS,)),
            pltpu.SemaphoreType.DMA((SLOTS,)),
            pltpu.SemaphoreType.REGULAR((SLOTS,)),
            pltpu.SemaphoreType.DMA((SLOTS,)),
            pltpu.SemaphoreType.DMA((SLOTS,)),
            pltpu.SemaphoreType.REGULAR((SLOTS,)),
        ],
        compiler_params=pltpu.CompilerParams(
            collective_id=0, vmem_limit_bytes=48 * 1024 * 1024
        ),
    )(partial)


# device time: 888538 ns/iter; 1.0021x vs baseline; 1.0021x over previous
import jax
import jax.numpy as jnp
from jax import lax
from jax.experimental import pallas as pl
from jax.experimental.pallas import tpu as pltpu

N_DEV = 32
M = 4096
N = 2048
MH = M // 2
CH = MH // N_DEV
SLOTS = 4
STEPS = N_DEV - 1


def kernel(x, w_mat):
    partial = jnp.dot(x, w_mat, preferred_element_type=jnp.float32)

    def body(
        p_ref,
        out_ref,
        comm_r,
        comm_l,
        stage_r,
        stage_l,
        local_sems,
        send_sems_r,
        recv_sems_r,
        credit_r,
        send_sems_l,
        recv_sems_l,
        credit_l,
        ag_send_r,
        ag_recv_r,
        ag_credit_r,
        ag_send_l,
        ag_recv_l,
        ag_credit_l,
    ):
        my = lax.axis_index("i")
        left = lax.rem(my - 1 + N_DEV, N_DEV)
        right = lax.rem(my + 1, N_DEV)

        barrier_sem = pltpu.get_barrier_semaphore()
        for nbr in (left, right):
            pl.semaphore_signal(
                barrier_sem,
                inc=1,
                device_id=(nbr,),
                device_id_type=pl.DeviceIdType.MESH,
            )
        pl.semaphore_wait(barrier_sem, 2)

        def rhalf(ref, c):
            return ref.at[pl.ds(c * CH, CH), :]

        def lhalf(ref, c):
            return ref.at[pl.ds(MH + c * CH, CH), :]

        def remote(src, dst, ssem, rsem, dev):
            return pltpu.make_async_remote_copy(
                src_ref=src,
                dst_ref=dst,
                send_sem=ssem,
                recv_sem=rsem,
                device_id=(dev,),
                device_id_type=pl.DeviceIdType.MESH,
            )

        def credit_to(sems, slot, dev):
            pl.semaphore_signal(
                sems.at[slot],
                inc=1,
                device_id=(dev,),
                device_id_type=pl.DeviceIdType.MESH,
            )

        seed_r = pltpu.make_async_copy(
            rhalf(p_ref, my), rhalf(out_ref, my), local_sems.at[0]
        )
        seed_l = pltpu.make_async_copy(
            lhalf(p_ref, my), lhalf(out_ref, my), local_sems.at[1]
        )
        seed_r.start()
        seed_l.start()
        seed_r.wait()
        seed_l.wait()

        def stage_issue(s):
            rc_r = lax.rem(my - s - 1 + 2 * N_DEV, N_DEV)
            rc_l = lax.rem(my + s + 1, N_DEV)
            sl = s % 2
            cr = pltpu.make_async_copy(
                rhalf(p_ref, rc_r), stage_r.at[sl], local_sems.at[sl]
            )
            cl = pltpu.make_async_copy(
                lhalf(p_ref, rc_l), stage_l.at[sl], local_sems.at[2 + sl]
            )
            cr.start()
            cl.start()
            return cr, cl

        stages = {0: stage_issue(0)}

        sends_r, sends_l = [], []
        for s in range(STEPS):
            slot = s % SLOTS
            sc_r = lax.rem(my - s + N_DEV, N_DEV)
            rc_r = lax.rem(my - s - 1 + 2 * N_DEV, N_DEV)
            sc_l = lax.rem(my + s, N_DEV)
            rc_l = lax.rem(my + s + 1, N_DEV)
            if s >= SLOTS:
                pl.semaphore_wait(credit_r.at[slot], 1)
                pl.semaphore_wait(credit_l.at[slot], 1)
                sends_r[s - SLOTS].wait_send()
                sends_l[s - SLOTS].wait_send()
            rdma_r = remote(
                rhalf(out_ref, sc_r),
                comm_r.at[slot],
                send_sems_r.at[slot],
                recv_sems_r.at[slot],
                right,
            )
            rdma_l = remote(
                lhalf(out_ref, sc_l),
                comm_l.at[slot],
                send_sems_l.at[slot],
                recv_sems_l.at[slot],
                left,
            )
            rdma_r.start()
            rdma_l.start()
            sends_r.append(rdma_r)
            sends_l.append(rdma_l)
            if s + 1 < STEPS:
                stages[s + 1] = stage_issue(s + 1)
            st_r, st_l = stages.pop(s)
            rdma_r.wait_recv()
            st_r.wait()
            acc = stage_r[s % 2] + comm_r[slot]
            if s == STEPS - 1:
                acc = acc * jax.nn.sigmoid(acc)
            out_ref[pl.ds(rc_r * CH, CH), :] = acc
            credit_to(credit_r, slot, left)
            rdma_l.wait_recv()
            st_l.wait()
            acc = stage_l[s % 2] + comm_l[slot]
            if s == STEPS - 1:
                acc = acc * jax.nn.sigmoid(acc)
            out_ref[pl.ds(MH + rc_l * CH, CH), :] = acc
            credit_to(credit_l, slot, right)
        for j in range(SLOTS):
            pl.semaphore_wait(credit_r.at[j], 1)
            pl.semaphore_wait(credit_l.at[j], 1)
        for d in sends_r[-SLOTS:] + sends_l[-SLOTS:]:
            d.wait_send()

        sends_r, sends_l = [], []
        for s in range(STEPS):
            slot = s % SLOTS
            sc_r = lax.rem(my + 1 - s + 2 * N_DEV, N_DEV)
            sc_l = lax.rem(my - 1 + s + N_DEV, N_DEV)
            if s >= SLOTS:
                pl.semaphore_wait(ag_credit_r.at[slot], 1)
                pl.semaphore_wait(ag_credit_l.at[slot], 1)
                sends_r[s - SLOTS].wait_send()
                sends_l[s - SLOTS].wait_send()
            rdma_r = remote(
                rhalf(out_ref, sc_r),
                rhalf(out_ref, sc_r),
                ag_send_r.at[slot],
                ag_recv_r.at[slot],
                right,
            )
            rdma_l = remote(
                lhalf(out_ref, sc_l),
                lhalf(out_ref, sc_l),
                ag_send_l.at[slot],
                ag_recv_l.at[slot],
                left,
            )
            rdma_r.start()
            rdma_l.start()
            sends_r.append(rdma_r)
            sends_l.append(rdma_l)
            rdma_r.wait_recv()
            credit_to(ag_credit_r, slot, left)
            rdma_l.wait_recv()
            credit_to(ag_credit_l, slot, right)
        for j in range(SLOTS):
            pl.semaphore_wait(ag_credit_r.at[j], 1)
            pl.semaphore_wait(ag_credit_l.at[j], 1)
        for d in sends_r[-SLOTS:] + sends_l[-SLOTS:]:
            d.wait_send()

    return pl.pallas_call(
        body,
        out_shape=jax.ShapeDtypeStruct((M, N), jnp.float32),
        in_specs=[pl.BlockSpec(memory_space=pltpu.MemorySpace.HBM)],
        out_specs=pl.BlockSpec(memory_space=pltpu.VMEM),
        scratch_shapes=[
            pltpu.VMEM((SLOTS, CH, N), jnp.float32),
            pltpu.VMEM((SLOTS, CH, N), jnp.float32),
            pltpu.VMEM((2, CH, N), jnp.float32),
            pltpu.VMEM((2, CH, N), jnp.float32),
            pltpu.SemaphoreType.DMA((4,)),
            pltpu.SemaphoreType.DMA((SLOTS,)),
            pltpu.SemaphoreType.DMA((SLOTS,)),
            pltpu.SemaphoreType.REGULAR((SLOTS,)),
            pltpu.SemaphoreType.DMA((SLOTS,)),
            pltpu.SemaphoreType.DMA((SLOTS,)),
            pltpu.SemaphoreType.REGULAR((SLOTS,)),
            pltpu.SemaphoreType.DMA((SLOTS,)),
            pltpu.SemaphoreType.DMA((SLOTS,)),
            pltpu.SemaphoreType.REGULAR((SLOTS,)),
            pltpu.SemaphoreType.DMA((SLOTS,)),
            pltpu.SemaphoreType.DMA((SLOTS,)),
            pltpu.SemaphoreType.REGULAR((SLOTS,)),
        ],
        compiler_params=pltpu.CompilerParams(
            collective_id=0, vmem_limit_bytes=48 * 1024 * 1024
        ),
    )(partial)
